# baseline (speedup 1.0000x reference)
import functools

import jax
import jax.numpy as jnp
from jax import lax
from jax.experimental import pallas as pl
from jax.experimental.pallas import tpu as pltpu
from jax.experimental.pallas import tpu_sc as plsc

N = 10000
NPAD = 10240
E = 320000
D = 128
HID2 = 256

NT = 32
CHUNK = 128
CPT = 80
IB = 40
NB = CPT // IB
EPAD = NT * CPT * CHUNK
RPT = NPAD // 16

BN = 1280
GRID = NPAD // BN
BNF = 1000
GRIDF = N // BNF
ER = E // CHUNK
ERP = EPAD // CHUNK
ERB = 320

_mesh = plsc.VectorSubcoreMesh(core_axis_name="c", subcore_axis_name="s")


def _edges_body(ei_ref, out_ref):
    j = pl.program_id(1)
    r_i = lax.broadcasted_iota(jnp.int32, (1, ERB, CHUNK), 1) + j * ERB
    l_i = lax.broadcasted_iota(jnp.int32, (1, ERB, CHUNK), 2)
    eid = r_i * CHUNK + l_i
    pad = N + lax.rem(eid, jnp.int32(NPAD - N))
    v = jnp.where(r_i < ER, ei_ref[...], pad)
    out_ref[...] = v.reshape(1, ERB // CPT, CPT, CHUNK)


_edges_call = pl.pallas_call(
    _edges_body,
    grid=(2, ERP // ERB),
    in_specs=[pl.BlockSpec((1, ERB, CHUNK), lambda i, j: (i, j, 0))],
    out_specs=pl.BlockSpec((1, ERB // CPT, CPT, CHUNK), lambda i, j: (i, j, 0, 0)),
    out_shape=jax.ShapeDtypeStruct((2, NT, CPT, CHUNK), jnp.int32),
)


@functools.partial(
    pl.kernel,
    mesh=_mesh,
    out_type=jax.ShapeDtypeStruct((2, NPAD, 16), jnp.float32),
    scratch_types=[
        pltpu.VMEM((CPT, CHUNK), jnp.int32),
        pltpu.VMEM((CHUNK, 16), jnp.float32),
        pltpu.VMEM((CHUNK, 16), jnp.float32),
        pltpu.VMEM_SHARED((NPAD, 16), jnp.float32),
        pltpu.SemaphoreType.DMA,
    ],
)
def _deg_kernel(edges_hbm, out_hbm, idx_v, ones_v, zb_v, deg_sh, hsem):
    c = lax.axis_index("c")
    s = lax.axis_index("s")
    w = c * 16 + s
    r0 = s * RPT

    def fill(i, carry):
        ones_v[i] = jnp.ones((16,), jnp.float32)
        zb_v[i] = jnp.zeros((16,), jnp.float32)
        return carry

    lax.fori_loop(0, CHUNK, fill, 0)
    for k in range(RPT // CHUNK):
        pltpu.sync_copy(zb_v, deg_sh.at[pl.ds(r0 + k * CHUNK, CHUNK)])
    pltpu.sync_copy(edges_hbm.at[1, w], idx_v)
    plsc.subcore_barrier()

    def fire(j, carry):
        pltpu.async_copy(ones_v, deg_sh.at[idx_v.at[j]], hsem, add=True)
        return carry

    lax.fori_loop(0, CPT, fire, 0)

    def drain(j, carry):
        pltpu.make_async_copy(ones_v, deg_sh.at[idx_v.at[0]], hsem).wait()
        return carry

    lax.fori_loop(0, CPT, drain, 0)
    plsc.subcore_barrier()
    pltpu.sync_copy(deg_sh.at[pl.ds(r0, RPT)], out_hbm.at[c, pl.ds(r0, RPT)])


@functools.partial(
    pl.kernel,
    mesh=_mesh,
    out_type=jax.ShapeDtypeStruct((2, NPAD, D), jnp.float32),
    scratch_types=[
        pltpu.VMEM((IB, CHUNK), jnp.int32),
        pltpu.VMEM((IB, CHUNK), jnp.int32),
        pltpu.VMEM((CHUNK, D), jnp.float32),
        pltpu.VMEM((CHUNK, D), jnp.float32),
        pltpu.VMEM_SHARED((NPAD, D), jnp.float32),
        pltpu.SemaphoreType.DMA,
        pltpu.SemaphoreType.DMA,
        pltpu.SemaphoreType.DMA,
        pltpu.SemaphoreType.DMA,
    ],
)
def _agg_kernel(g_hbm, edges_hbm, out_hbm,
                src_v, dst_v, buf_a, buf_b, acc_sh, sem_a, sem_b,
                ssem_a, ssem_b):
    c = lax.axis_index("c")
    s = lax.axis_index("s")
    w = c * 16 + s
    r0 = s * RPT

    def zfill(i, carry):
        for k in range(D // 16):
            buf_a[i, pl.ds(k * 16, 16)] = jnp.zeros((16,), jnp.float32)
        return carry

    lax.fori_loop(0, CHUNK, zfill, 0)
    for k in range(RPT // CHUNK):
        pltpu.sync_copy(buf_a, acc_sh.at[pl.ds(r0 + k * CHUNK, CHUNK)])
    plsc.subcore_barrier()

    def block(b, carry):
        pltpu.sync_copy(edges_hbm.at[0, w, pl.ds(b * IB, IB)], src_v)
        pltpu.sync_copy(edges_hbm.at[1, w, pl.ds(b * IB, IB)], dst_v)
        pltpu.async_copy(g_hbm.at[src_v.at[0]], buf_a, sem_a)

        def body(jj, c2):
            j0 = 2 * jj
            j1 = j0 + 1
            pltpu.make_async_copy(g_hbm.at[src_v.at[j0]], buf_a, sem_a).wait()
            pltpu.async_copy(g_hbm.at[src_v.at[j1]], buf_b, sem_b)
            pltpu.async_copy(buf_a, acc_sh.at[dst_v.at[j0]], ssem_a, add=True)
            pltpu.make_async_copy(g_hbm.at[src_v.at[j1]], buf_b, sem_b).wait()
            pltpu.async_copy(buf_b, acc_sh.at[dst_v.at[j1]], ssem_b, add=True)
            pltpu.make_async_copy(buf_a, acc_sh.at[dst_v.at[j0]], ssem_a).wait()

            @pl.when(jj < IB // 2 - 1)
            def _prefetch():
                pltpu.async_copy(g_hbm.at[src_v.at[j0 + 2]], buf_a, sem_a)

            pltpu.make_async_copy(buf_b, acc_sh.at[dst_v.at[j1]], ssem_b).wait()
            return c2

        lax.fori_loop(0, IB // 2, body, 0)
        return carry

    lax.fori_loop(0, NB, block, 0)
    plsc.subcore_barrier()
    pltpu.sync_copy(acc_sh.at[pl.ds(r0, RPT)], out_hbm.at[c, pl.ds(r0, RPT)])


def _dis_of(deg_ref):
    d = deg_ref[...]
    return lax.rsqrt(1.0 + d[0, :, 0:1] + d[1, :, 0:1])


def _pre_body(deg_ref, x_ref, g0_ref):
    rows = pl.program_id(0) * BN + lax.broadcasted_iota(jnp.int32, (BN, 1), 0)
    g0 = x_ref[...] * _dis_of(deg_ref)
    g0_ref[...] = jnp.where(rows < N, g0, 0.0)


_pre_call = pl.pallas_call(
    _pre_body,
    grid=(GRID,),
    in_specs=[
        pl.BlockSpec((2, BN, 16), lambda i: (0, i, 0)),
        pl.BlockSpec((BN, D), lambda i: (i, 0)),
    ],
    out_specs=pl.BlockSpec((BN, D), lambda i: (i, 0)),
    out_shape=jax.ShapeDtypeStruct((NPAD, D), jnp.float32),
)


def _mid_body(deg_ref, p_ref, g0_ref, w1_ref, b1_ref, w2_ref, g1_ref):
    dis = _dis_of(deg_ref)
    p = p_ref[...]
    a = (p[0] + p[1] + g0_ref[...]) * dis
    h = jnp.dot(a, w1_ref[...], preferred_element_type=jnp.float32)
    h = jnp.maximum(h + b1_ref[...], 0.0)
    g1 = jnp.dot(h, w2_ref[...], preferred_element_type=jnp.float32) * dis
    rows = pl.program_id(0) * BN + lax.broadcasted_iota(jnp.int32, (BN, 1), 0)
    g1_ref[...] = jnp.where(rows < N, g1, 0.0)


_mid_call = pl.pallas_call(
    _mid_body,
    grid=(GRID,),
    in_specs=[
        pl.BlockSpec((2, BN, 16), lambda i: (0, i, 0)),
        pl.BlockSpec((2, BN, D), lambda i: (0, i, 0)),
        pl.BlockSpec((BN, D), lambda i: (i, 0)),
        pl.BlockSpec((D, HID2), lambda i: (0, 0)),
        pl.BlockSpec((1, HID2), lambda i: (0, 0)),
        pl.BlockSpec((HID2, D), lambda i: (0, 0)),
    ],
    out_specs=pl.BlockSpec((BN, D), lambda i: (i, 0)),
    out_shape=jax.ShapeDtypeStruct((NPAD, D), jnp.float32),
)


def _fin_body(deg_ref, p_ref, g1_ref, b2_ref, wl_ref, bl_ref, y_ref):
    dis = _dis_of(deg_ref)
    p = p_ref[...]
    p2 = (p[0] + p[1] + g1_ref[...]) * dis + b2_ref[...]
    p2 = jnp.maximum(p2, 0.0)
    y_ref[...] = jnp.dot(p2, wl_ref[...], preferred_element_type=jnp.float32) + bl_ref[...]


_fin_call = pl.pallas_call(
    _fin_body,
    grid=(GRIDF,),
    in_specs=[
        pl.BlockSpec((2, BNF, 16), lambda i: (0, i, 0)),
        pl.BlockSpec((2, BNF, D), lambda i: (0, i, 0)),
        pl.BlockSpec((BNF, D), lambda i: (i, 0)),
        pl.BlockSpec((1, D), lambda i: (0, 0)),
        pl.BlockSpec((D, D), lambda i: (0, 0)),
        pl.BlockSpec((1, D), lambda i: (0, 0)),
    ],
    out_specs=pl.BlockSpec((BNF, D), lambda i: (i, 0)),
    out_shape=jax.ShapeDtypeStruct((N, D), jnp.float32),
)


def kernel(x, edge_attrs, edge_index, W1, b1, W2, b2, Wl, bl):
    del edge_attrs
    edges = _edges_call(edge_index.reshape(2, ER, CHUNK))
    degp = _deg_kernel(edges)
    g0 = _pre_call(degp, x)
    part = _agg_kernel(g0, edges)
    g1 = _mid_call(degp, part, g0, W1, b1.reshape(1, -1), W2)
    part2 = _agg_kernel(g1, edges)
    return _fin_call(degp, part2, g1, b2.reshape(1, -1), Wl, bl.reshape(1, -1))


# speedup vs baseline: 1.0088x; 1.0088x over previous
import functools

import jax
import jax.numpy as jnp
from jax import lax
from jax.experimental import pallas as pl
from jax.experimental.pallas import tpu as pltpu
from jax.experimental.pallas import tpu_sc as plsc

N = 10000
NPAD = 10240
E = 320000
D = 128
HID2 = 256

NT = 32
CHUNK = 128
CPT = 80
IB = 40
NB = CPT // IB
EPAD = NT * CPT * CHUNK
RPT = NPAD // 16

BN = 1280
GRID = NPAD // BN
BNF = 1000
GRIDF = N // BNF
ER = E // CHUNK
ERP = EPAD // CHUNK
ERB = 640

_mesh = plsc.VectorSubcoreMesh(core_axis_name="c", subcore_axis_name="s")


def _edges_body(ei_ref, out_ref):
    j = pl.program_id(1)
    r_i = lax.broadcasted_iota(jnp.int32, (1, ERB, CHUNK), 1) + j * ERB
    l_i = lax.broadcasted_iota(jnp.int32, (1, ERB, CHUNK), 2)
    eid = r_i * CHUNK + l_i
    pad = N + lax.rem(eid, jnp.int32(NPAD - N))
    v = jnp.where(r_i < ER, ei_ref[...], pad)
    out_ref[...] = v.reshape(1, ERB // CPT, CPT, CHUNK)


_edges_call = pl.pallas_call(
    _edges_body,
    grid=(2, ERP // ERB),
    in_specs=[pl.BlockSpec((1, ERB, CHUNK), lambda i, j: (i, j, 0))],
    out_specs=pl.BlockSpec((1, ERB // CPT, CPT, CHUNK), lambda i, j: (i, j, 0, 0)),
    out_shape=jax.ShapeDtypeStruct((2, NT, CPT, CHUNK), jnp.int32),
)


@functools.partial(
    pl.kernel,
    mesh=_mesh,
    out_type=jax.ShapeDtypeStruct((2, NPAD, 16), jnp.float32),
    scratch_types=[
        pltpu.VMEM((CPT, CHUNK), jnp.int32),
        pltpu.VMEM((CHUNK, 16), jnp.float32),
        pltpu.VMEM((CHUNK, 16), jnp.float32),
        pltpu.VMEM_SHARED((NPAD, 16), jnp.float32),
        pltpu.SemaphoreType.DMA,
    ],
)
def _deg_kernel(edges_hbm, out_hbm, idx_v, ones_v, zb_v, deg_sh, hsem):
    c = lax.axis_index("c")
    s = lax.axis_index("s")
    w = c * 16 + s
    r0 = s * RPT

    def fill(i, carry):
        ones_v[i] = jnp.ones((16,), jnp.float32)
        zb_v[i] = jnp.zeros((16,), jnp.float32)
        return carry

    lax.fori_loop(0, CHUNK, fill, 0)
    for k in range(RPT // CHUNK):
        pltpu.sync_copy(zb_v, deg_sh.at[pl.ds(r0 + k * CHUNK, CHUNK)])
    pltpu.sync_copy(edges_hbm.at[1, w], idx_v)
    plsc.subcore_barrier()

    def fire(j, carry):
        pltpu.async_copy(ones_v, deg_sh.at[idx_v.at[j]], hsem, add=True)
        return carry

    lax.fori_loop(0, CPT, fire, 0)

    def drain(j, carry):
        pltpu.make_async_copy(ones_v, deg_sh.at[idx_v.at[0]], hsem).wait()
        return carry

    lax.fori_loop(0, CPT, drain, 0)
    plsc.subcore_barrier()
    pltpu.sync_copy(deg_sh.at[pl.ds(r0, RPT)], out_hbm.at[c, pl.ds(r0, RPT)])


@functools.partial(
    pl.kernel,
    mesh=_mesh,
    out_type=jax.ShapeDtypeStruct((2, NPAD, D), jnp.float32),
    scratch_types=[
        pltpu.VMEM((IB, CHUNK), jnp.int32),
        pltpu.VMEM((IB, CHUNK), jnp.int32),
        pltpu.VMEM((CHUNK, D), jnp.float32),
        pltpu.VMEM((CHUNK, D), jnp.float32),
        pltpu.VMEM_SHARED((NPAD, D), jnp.float32),
        pltpu.SemaphoreType.DMA,
        pltpu.SemaphoreType.DMA,
        pltpu.SemaphoreType.DMA,
        pltpu.SemaphoreType.DMA,
    ],
)
def _agg_kernel(g_hbm, edges_hbm, out_hbm,
                src_v, dst_v, buf_a, buf_b, acc_sh, sem_a, sem_b,
                ssem_a, ssem_b):
    c = lax.axis_index("c")
    s = lax.axis_index("s")
    w = c * 16 + s
    r0 = s * RPT

    def zfill(i, carry):
        for k in range(D // 16):
            buf_a[i, pl.ds(k * 16, 16)] = jnp.zeros((16,), jnp.float32)
        return carry

    lax.fori_loop(0, CHUNK, zfill, 0)
    for k in range(RPT // CHUNK):
        pltpu.sync_copy(buf_a, acc_sh.at[pl.ds(r0 + k * CHUNK, CHUNK)])
    plsc.subcore_barrier()

    def block(b, carry):
        pltpu.sync_copy(edges_hbm.at[0, w, pl.ds(b * IB, IB)], src_v)
        pltpu.sync_copy(edges_hbm.at[1, w, pl.ds(b * IB, IB)], dst_v)
        pltpu.async_copy(g_hbm.at[src_v.at[0]], buf_a, sem_a)

        def body(jj, c2):
            j0 = 2 * jj
            j1 = j0 + 1
            pltpu.make_async_copy(g_hbm.at[src_v.at[j0]], buf_a, sem_a).wait()
            pltpu.async_copy(g_hbm.at[src_v.at[j1]], buf_b, sem_b)
            pltpu.async_copy(buf_a, acc_sh.at[dst_v.at[j0]], ssem_a, add=True)
            pltpu.make_async_copy(g_hbm.at[src_v.at[j1]], buf_b, sem_b).wait()
            pltpu.async_copy(buf_b, acc_sh.at[dst_v.at[j1]], ssem_b, add=True)
            pltpu.make_async_copy(buf_a, acc_sh.at[dst_v.at[j0]], ssem_a).wait()

            @pl.when(jj < IB // 2 - 1)
            def _prefetch():
                pltpu.async_copy(g_hbm.at[src_v.at[j0 + 2]], buf_a, sem_a)

            pltpu.make_async_copy(buf_b, acc_sh.at[dst_v.at[j1]], ssem_b).wait()
            return c2

        lax.fori_loop(0, IB // 2, body, 0)
        return carry

    lax.fori_loop(0, NB, block, 0)
    plsc.subcore_barrier()
    pltpu.sync_copy(acc_sh.at[pl.ds(r0, RPT)], out_hbm.at[c, pl.ds(r0, RPT)])


def _dis_of(deg_ref):
    d = deg_ref[...]
    return lax.rsqrt(1.0 + d[0, :, 0:1] + d[1, :, 0:1])


def _pre_body(deg_ref, x_ref, g0_ref):
    rows = pl.program_id(0) * BN + lax.broadcasted_iota(jnp.int32, (BN, 1), 0)
    g0 = x_ref[...] * _dis_of(deg_ref)
    g0_ref[...] = jnp.where(rows < N, g0, 0.0)


_pre_call = pl.pallas_call(
    _pre_body,
    grid=(GRID,),
    in_specs=[
        pl.BlockSpec((2, BN, 16), lambda i: (0, i, 0)),
        pl.BlockSpec((BN, D), lambda i: (i, 0)),
    ],
    out_specs=pl.BlockSpec((BN, D), lambda i: (i, 0)),
    out_shape=jax.ShapeDtypeStruct((NPAD, D), jnp.float32),
)


def _mid_body(deg_ref, p_ref, g0_ref, w1_ref, b1_ref, w2_ref, g1_ref):
    dis = _dis_of(deg_ref)
    p = p_ref[...]
    a = (p[0] + p[1] + g0_ref[...]) * dis
    h = jnp.dot(a, w1_ref[...], preferred_element_type=jnp.float32)
    h = jnp.maximum(h + b1_ref[...], 0.0)
    g1 = jnp.dot(h, w2_ref[...], preferred_element_type=jnp.float32) * dis
    rows = pl.program_id(0) * BN + lax.broadcasted_iota(jnp.int32, (BN, 1), 0)
    g1_ref[...] = jnp.where(rows < N, g1, 0.0)


_mid_call = pl.pallas_call(
    _mid_body,
    grid=(GRID,),
    in_specs=[
        pl.BlockSpec((2, BN, 16), lambda i: (0, i, 0)),
        pl.BlockSpec((2, BN, D), lambda i: (0, i, 0)),
        pl.BlockSpec((BN, D), lambda i: (i, 0)),
        pl.BlockSpec((D, HID2), lambda i: (0, 0)),
        pl.BlockSpec((1, HID2), lambda i: (0, 0)),
        pl.BlockSpec((HID2, D), lambda i: (0, 0)),
    ],
    out_specs=pl.BlockSpec((BN, D), lambda i: (i, 0)),
    out_shape=jax.ShapeDtypeStruct((NPAD, D), jnp.float32),
)


def _fin_body(deg_ref, p_ref, g1_ref, b2_ref, wl_ref, bl_ref, y_ref):
    dis = _dis_of(deg_ref)
    p = p_ref[...]
    p2 = (p[0] + p[1] + g1_ref[...]) * dis + b2_ref[...]
    p2 = jnp.maximum(p2, 0.0)
    y_ref[...] = jnp.dot(p2, wl_ref[...], preferred_element_type=jnp.float32) + bl_ref[...]


_fin_call = pl.pallas_call(
    _fin_body,
    grid=(GRIDF,),
    in_specs=[
        pl.BlockSpec((2, BNF, 16), lambda i: (0, i, 0)),
        pl.BlockSpec((2, BNF, D), lambda i: (0, i, 0)),
        pl.BlockSpec((BNF, D), lambda i: (i, 0)),
        pl.BlockSpec((1, D), lambda i: (0, 0)),
        pl.BlockSpec((D, D), lambda i: (0, 0)),
        pl.BlockSpec((1, D), lambda i: (0, 0)),
    ],
    out_specs=pl.BlockSpec((BNF, D), lambda i: (i, 0)),
    out_shape=jax.ShapeDtypeStruct((N, D), jnp.float32),
)


def kernel(x, edge_attrs, edge_index, W1, b1, W2, b2, Wl, bl):
    del edge_attrs
    edges = _edges_call(edge_index.reshape(2, ER, CHUNK))
    degp = _deg_kernel(edges)
    g0 = _pre_call(degp, x)
    part = _agg_kernel(g0, edges)
    g1 = _mid_call(degp, part, g0, W1, b1.reshape(1, -1), W2)
    part2 = _agg_kernel(g1, edges)
    return _fin_call(degp, part2, g1, b2.reshape(1, -1), Wl, bl.reshape(1, -1))


# speedup vs baseline: 1.0170x; 1.0081x over previous
import functools

import jax
import jax.numpy as jnp
from jax import lax
from jax.experimental import pallas as pl
from jax.experimental.pallas import tpu as pltpu
from jax.experimental.pallas import tpu_sc as plsc

N = 10000
NPAD = 10240
E = 320000
D = 128
HID2 = 256

NT = 32
CHUNK = 128
CPT = 80
IB = 40
NB = CPT // IB
EPAD = NT * CPT * CHUNK
RPT = NPAD // 16

BN = 1280
GRID = NPAD // BN
BNF = 1000
GRIDF = N // BNF
ER = E // CHUNK
ERP = EPAD // CHUNK
ERB = 640

_mesh = plsc.VectorSubcoreMesh(core_axis_name="c", subcore_axis_name="s")


EB = ERB * CHUNK


def _edges_body(ei_ref, out_ref):
    j = pl.program_id(0)
    eid = lax.broadcasted_iota(jnp.int32, (2, EB), 1) + j * EB
    pad = N + lax.rem(eid, jnp.int32(NPAD - N))
    v = jnp.where(eid < E, ei_ref[...], pad)
    out_ref[...] = v.reshape(2, ERB // CPT, CPT, CHUNK)


_edges_call = pl.pallas_call(
    _edges_body,
    grid=(ERP // ERB,),
    in_specs=[pl.BlockSpec((2, EB), lambda j: (0, j))],
    out_specs=pl.BlockSpec((2, ERB // CPT, CPT, CHUNK), lambda j: (0, j, 0, 0)),
    out_shape=jax.ShapeDtypeStruct((2, NT, CPT, CHUNK), jnp.int32),
)


@functools.partial(
    pl.kernel,
    mesh=_mesh,
    out_type=jax.ShapeDtypeStruct((2, NPAD, 16), jnp.float32),
    scratch_types=[
        pltpu.VMEM((CPT, CHUNK), jnp.int32),
        pltpu.VMEM((CHUNK, 16), jnp.float32),
        pltpu.VMEM((CHUNK, 16), jnp.float32),
        pltpu.VMEM_SHARED((NPAD, 16), jnp.float32),
        pltpu.SemaphoreType.DMA,
    ],
)
def _deg_kernel(edges_hbm, out_hbm, idx_v, ones_v, zb_v, deg_sh, hsem):
    c = lax.axis_index("c")
    s = lax.axis_index("s")
    w = c * 16 + s
    r0 = s * RPT

    def fill(i, carry):
        ones_v[i] = jnp.ones((16,), jnp.float32)
        zb_v[i] = jnp.zeros((16,), jnp.float32)
        return carry

    lax.fori_loop(0, CHUNK, fill, 0)
    for k in range(RPT // CHUNK):
        pltpu.sync_copy(zb_v, deg_sh.at[pl.ds(r0 + k * CHUNK, CHUNK)])
    pltpu.sync_copy(edges_hbm.at[1, w], idx_v)
    plsc.subcore_barrier()

    def fire(j, carry):
        pltpu.async_copy(ones_v, deg_sh.at[idx_v.at[j]], hsem, add=True)
        return carry

    lax.fori_loop(0, CPT, fire, 0)

    def drain(j, carry):
        pltpu.make_async_copy(ones_v, deg_sh.at[idx_v.at[0]], hsem).wait()
        return carry

    lax.fori_loop(0, CPT, drain, 0)
    plsc.subcore_barrier()
    pltpu.sync_copy(deg_sh.at[pl.ds(r0, RPT)], out_hbm.at[c, pl.ds(r0, RPT)])


@functools.partial(
    pl.kernel,
    mesh=_mesh,
    out_type=jax.ShapeDtypeStruct((2, NPAD, D), jnp.float32),
    scratch_types=[
        pltpu.VMEM((IB, CHUNK), jnp.int32),
        pltpu.VMEM((IB, CHUNK), jnp.int32),
        pltpu.VMEM((CHUNK, D), jnp.float32),
        pltpu.VMEM((CHUNK, D), jnp.float32),
        pltpu.VMEM_SHARED((NPAD, D), jnp.float32),
        pltpu.SemaphoreType.DMA,
        pltpu.SemaphoreType.DMA,
        pltpu.SemaphoreType.DMA,
        pltpu.SemaphoreType.DMA,
    ],
)
def _agg_kernel(g_hbm, edges_hbm, out_hbm,
                src_v, dst_v, buf_a, buf_b, acc_sh, sem_a, sem_b,
                ssem_a, ssem_b):
    c = lax.axis_index("c")
    s = lax.axis_index("s")
    w = c * 16 + s
    r0 = s * RPT

    def zfill(i, carry):
        for k in range(D // 16):
            buf_a[i, pl.ds(k * 16, 16)] = jnp.zeros((16,), jnp.float32)
        return carry

    lax.fori_loop(0, CHUNK, zfill, 0)
    for k in range(RPT // CHUNK):
        pltpu.sync_copy(buf_a, acc_sh.at[pl.ds(r0 + k * CHUNK, CHUNK)])
    plsc.subcore_barrier()

    def block(b, carry):
        pltpu.sync_copy(edges_hbm.at[0, w, pl.ds(b * IB, IB)], src_v)
        pltpu.sync_copy(edges_hbm.at[1, w, pl.ds(b * IB, IB)], dst_v)
        pltpu.async_copy(g_hbm.at[src_v.at[0]], buf_a, sem_a)

        def body(jj, c2):
            j0 = 2 * jj
            j1 = j0 + 1
            pltpu.make_async_copy(g_hbm.at[src_v.at[j0]], buf_a, sem_a).wait()
            pltpu.async_copy(g_hbm.at[src_v.at[j1]], buf_b, sem_b)
            pltpu.async_copy(buf_a, acc_sh.at[dst_v.at[j0]], ssem_a, add=True)
            pltpu.make_async_copy(g_hbm.at[src_v.at[j1]], buf_b, sem_b).wait()
            pltpu.async_copy(buf_b, acc_sh.at[dst_v.at[j1]], ssem_b, add=True)
            pltpu.make_async_copy(buf_a, acc_sh.at[dst_v.at[j0]], ssem_a).wait()

            @pl.when(jj < IB // 2 - 1)
            def _prefetch():
                pltpu.async_copy(g_hbm.at[src_v.at[j0 + 2]], buf_a, sem_a)

            pltpu.make_async_copy(buf_b, acc_sh.at[dst_v.at[j1]], ssem_b).wait()
            return c2

        lax.fori_loop(0, IB // 2, body, 0)
        return carry

    lax.fori_loop(0, NB, block, 0)
    plsc.subcore_barrier()
    pltpu.sync_copy(acc_sh.at[pl.ds(r0, RPT)], out_hbm.at[c, pl.ds(r0, RPT)])


def _dis_of(deg_ref):
    d = deg_ref[...]
    return lax.rsqrt(1.0 + d[0, :, 0:1] + d[1, :, 0:1])


def _pre_body(deg_ref, x_ref, g0_ref):
    rows = pl.program_id(0) * BN + lax.broadcasted_iota(jnp.int32, (BN, 1), 0)
    g0 = x_ref[...] * _dis_of(deg_ref)
    g0_ref[...] = jnp.where(rows < N, g0, 0.0)


_pre_call = pl.pallas_call(
    _pre_body,
    grid=(GRID,),
    in_specs=[
        pl.BlockSpec((2, BN, 16), lambda i: (0, i, 0)),
        pl.BlockSpec((BN, D), lambda i: (i, 0)),
    ],
    out_specs=pl.BlockSpec((BN, D), lambda i: (i, 0)),
    out_shape=jax.ShapeDtypeStruct((NPAD, D), jnp.float32),
)


def _mid_body(deg_ref, p_ref, g0_ref, w1_ref, b1_ref, w2_ref, g1_ref):
    dis = _dis_of(deg_ref)
    p = p_ref[...]
    a = (p[0] + p[1] + g0_ref[...]) * dis
    h = jnp.dot(a, w1_ref[...], preferred_element_type=jnp.float32)
    h = jnp.maximum(h + b1_ref[...], 0.0)
    g1 = jnp.dot(h, w2_ref[...], preferred_element_type=jnp.float32) * dis
    rows = pl.program_id(0) * BN + lax.broadcasted_iota(jnp.int32, (BN, 1), 0)
    g1_ref[...] = jnp.where(rows < N, g1, 0.0)


_mid_call = pl.pallas_call(
    _mid_body,
    grid=(GRID,),
    in_specs=[
        pl.BlockSpec((2, BN, 16), lambda i: (0, i, 0)),
        pl.BlockSpec((2, BN, D), lambda i: (0, i, 0)),
        pl.BlockSpec((BN, D), lambda i: (i, 0)),
        pl.BlockSpec((D, HID2), lambda i: (0, 0)),
        pl.BlockSpec((1, HID2), lambda i: (0, 0)),
        pl.BlockSpec((HID2, D), lambda i: (0, 0)),
    ],
    out_specs=pl.BlockSpec((BN, D), lambda i: (i, 0)),
    out_shape=jax.ShapeDtypeStruct((NPAD, D), jnp.float32),
)


def _fin_body(deg_ref, p_ref, g1_ref, b2_ref, wl_ref, bl_ref, y_ref):
    dis = _dis_of(deg_ref)
    p = p_ref[...]
    p2 = (p[0] + p[1] + g1_ref[...]) * dis + b2_ref[...]
    p2 = jnp.maximum(p2, 0.0)
    y_ref[...] = jnp.dot(p2, wl_ref[...], preferred_element_type=jnp.float32) + bl_ref[...]


_fin_call = pl.pallas_call(
    _fin_body,
    grid=(GRIDF,),
    in_specs=[
        pl.BlockSpec((2, BNF, 16), lambda i: (0, i, 0)),
        pl.BlockSpec((2, BNF, D), lambda i: (0, i, 0)),
        pl.BlockSpec((BNF, D), lambda i: (i, 0)),
        pl.BlockSpec((1, D), lambda i: (0, 0)),
        pl.BlockSpec((D, D), lambda i: (0, 0)),
        pl.BlockSpec((1, D), lambda i: (0, 0)),
    ],
    out_specs=pl.BlockSpec((BNF, D), lambda i: (i, 0)),
    out_shape=jax.ShapeDtypeStruct((N, D), jnp.float32),
)


def kernel(x, edge_attrs, edge_index, W1, b1, W2, b2, Wl, bl):
    del edge_attrs
    edges = _edges_call(edge_index)
    degp = _deg_kernel(edges)
    g0 = _pre_call(degp, x)
    part = _agg_kernel(g0, edges)
    g1 = _mid_call(degp, part, g0, W1, b1.reshape(1, -1), W2)
    part2 = _agg_kernel(g1, edges)
    return _fin_call(degp, part2, g1, b2.reshape(1, -1), Wl, bl.reshape(1, -1))
